# baseline (speedup 1.0000x reference)
import dataclasses
import functools

import jax
import jax.numpy as jnp
from jax import lax
from jax.experimental import pallas as pl
from jax.experimental.pallas import tpu as pltpu
from jax.experimental.pallas import tpu_sc as plsc

MIN_RADIUS = 0.01
MAX_RADIUS = 1.0

LANES = 16
NUM_WORKERS = 32
GRP = 256


def _radii_coef_kernel(lr_ref, out_ref):
    r = jnp.clip(jnp.exp(lr_ref[...]), MIN_RADIUS, MAX_RADIUS)
    out_ref[...] = -1.0 / (2.0 * r * r + 1e-8)


def _make_sc_kernel(M, K, N, MAIN):
    if MAIN % GRP:
        raise ValueError("aligned prefix must divide the group size")
    NFULL = MAIN // GRP
    TMAX = -(-NFULL // NUM_WORKERS)
    UMAX = -(-TMAX // 2)
    mesh = plsc.VectorSubcoreMesh(
        core_axis_name="c", subcore_axis_name="s", num_cores=2, num_subcores=16
    )
    cp = pltpu.CompilerParams()
    if "needs_layout_passes" in pltpu.CompilerParams.__dataclass_fields__:
        cp = dataclasses.replace(cp, needs_layout_passes=False)

    @functools.partial(
        pl.kernel,
        out_type=(
            jax.ShapeDtypeStruct((K, M), jnp.float32),
            jax.ShapeDtypeStruct((K, 128), jnp.float32),
        ),
        mesh=mesh,
        compiler_params=cp,
        scratch_types=[
            pltpu.VMEM((3, N), jnp.float32),
            pltpu.VMEM((N,), jnp.float32),
            pltpu.VMEM((K, GRP), jnp.int32),
            pltpu.VMEM((K, GRP), jnp.int32),
            pltpu.VMEM((3, GRP), jnp.float32),
            pltpu.VMEM((3, GRP), jnp.float32),
            pltpu.VMEM((K, GRP), jnp.float32),
            pltpu.VMEM((K, GRP), jnp.float32),
            pltpu.SemaphoreType.DMA,
            pltpu.SemaphoreType.DMA,
            pltpu.SemaphoreType.DMA,
            pltpu.SemaphoreType.DMA,
            pltpu.SemaphoreType.DMA,
            pltpu.SemaphoreType.DMA,
        ],
    )
    def sc_kernel(q_hbm, i_hbm, qt_hbm, it_hbm, p_hbm, c_hbm, o_hbm, ot_hbm,
                  tpos, tcoef, ib0, ib1, qb0, qb1, ob0, ob1,
                  is0, is1, qs0, qs1, os0, os1):
        wid = lax.axis_index("s") * 2 + lax.axis_index("c")
        ibuf, qbuf, obuf = (ib0, ib1), (qb0, qb1), (ob0, ob1)
        isem, qsem, osem = (is0, is1), (qs0, qs1), (os0, os1)
        tp_desc = pltpu.async_copy(p_hbm, tpos, os0)
        tc_desc = pltpu.async_copy(c_hbm, tcoef, os1)

        def compute(ib, qb, ob, width):
            @plsc.parallel_loop(0, width // LANES, unroll=4)
            def _lane_group(l):
                lane = lax.iota(jnp.int32, LANES) + l * LANES

                def row(k):
                    return jnp.full((LANES,), k, jnp.int32)

                qx = plsc.load_gather(qb, [row(0), lane])
                qy = plsc.load_gather(qb, [row(1), lane])
                qz = plsc.load_gather(qb, [row(2), lane])
                acc = jnp.full((LANES,), 1e-8, jnp.float32)
                ws = []
                for k in range(K):
                    idxv = plsc.load_gather(ib, [row(k), lane])
                    px = plsc.load_gather(tpos, [row(0), idxv])
                    py = plsc.load_gather(tpos, [row(1), idxv])
                    pz = plsc.load_gather(tpos, [row(2), idxv])
                    cc = plsc.load_gather(tcoef, [idxv])
                    dx = px - qx
                    dy = py - qy
                    dz = pz - qz
                    d2 = dx * dx + dy * dy + dz * dz
                    w = jnp.exp(d2 * cc)
                    acc = acc + w
                    ws.append(w)
                r = jnp.full((LANES,), 1.0, jnp.float32) / acc
                for k in range(K):
                    plsc.store_scatter(ob, [row(k), lane], ws[k] * r)

        def start_in(g, b):
            m0 = g * GRP
            pltpu.async_copy(i_hbm.at[:, pl.ds(m0, GRP)], ibuf[b], isem[b])
            pltpu.async_copy(q_hbm.at[:, pl.ds(m0, GRP)], qbuf[b], qsem[b])

        def wait_in(b):
            pltpu.make_async_copy(i_hbm.at[:, pl.ds(0, GRP)], ibuf[b], isem[b]).wait()
            pltpu.make_async_copy(q_hbm.at[:, pl.ds(0, GRP)], qbuf[b], qsem[b]).wait()

        def wait_out(b):
            pltpu.make_async_copy(obuf[b], o_hbm.at[:, pl.ds(0, GRP)], osem[b]).wait()

        start_in(wid, 0)
        tp_desc.wait()
        tc_desc.wait()

        @pl.loop(0, UMAX)
        def _pair(u):
            t0 = u * 2
            for b in (0, 1):
                t = t0 + b
                g = wid + t * NUM_WORKERS

                @pl.when(g < NFULL)
                def _phase():
                    gn = g + NUM_WORKERS

                    @pl.when(gn < NFULL)
                    def _prefetch():
                        start_in(gn, 1 - b)

                    wait_in(b)

                    @pl.when(t >= 2)
                    def _drain_prev():
                        wait_out(b)

                    compute(ibuf[b], qbuf[b], obuf[b], GRP)
                    pltpu.async_copy(
                        obuf[b], o_hbm.at[:, pl.ds(g * GRP, GRP)], osem[b])

        nmine = (NFULL - wid + NUM_WORKERS - 1) // NUM_WORKERS
        for b in (0, 1):
            @pl.when(nmine >= b + 1)
            def _drain():
                wait_out(b)

        @pl.when(wid == 0)
        def _tail():
            pltpu.sync_copy(it_hbm, ib0.at[:, pl.ds(0, 128)])
            pltpu.sync_copy(qt_hbm, qb0.at[:, pl.ds(0, 128)])
            compute(ib0, qb0, ob0, 128)
            pltpu.sync_copy(ob0.at[:, pl.ds(0, 128)], ot_hbm)

    return sc_kernel


def kernel(query_points, k_nearest_indices, positions, log_radii):
    M, K = k_nearest_indices.shape
    N = positions.shape[0]
    TAIL = M % 128
    MAIN = M - TAIL

    coef = pl.pallas_call(
        _radii_coef_kernel,
        out_shape=jax.ShapeDtypeStruct(log_radii.shape, jnp.float32),
    )(log_radii.astype(jnp.float32))

    q_t = query_points.astype(jnp.float32).T
    i_t = k_nearest_indices.astype(jnp.int32).T
    q_tail = lax.dynamic_slice(q_t, (0, M - 128), (3, 128))
    i_tail = lax.dynamic_slice(i_t, (0, M - 128), (K, 128))

    out_t, out_tail = _make_sc_kernel(M, K, N, MAIN)(
        q_t,
        i_t,
        q_tail,
        i_tail,
        positions.astype(jnp.float32).T,
        coef.reshape(-1),
    )
    if TAIL:
        out_t = lax.dynamic_update_slice(out_t, out_tail, (0, M - 128))
    return out_t.T


# speedup vs baseline: 1.1110x; 1.1110x over previous
import dataclasses
import functools

import jax
import jax.numpy as jnp
from jax import lax
from jax.experimental import pallas as pl
from jax.experimental.pallas import tpu as pltpu
from jax.experimental.pallas import tpu_sc as plsc

MIN_RADIUS = 0.01
MAX_RADIUS = 1.0

LANES = 16
NUM_WORKERS = 32
GRP = 512


def _radii_coef_kernel(lr_ref, out_ref):
    r = jnp.clip(jnp.exp(lr_ref[...]), MIN_RADIUS, MAX_RADIUS)
    out_ref[...] = -1.0 / (2.0 * r * r + 1e-8)


def _make_sc_kernel(M, K, N, MAIN):
    if MAIN % GRP:
        raise ValueError("aligned prefix must divide the group size")
    NFULL = MAIN // GRP
    TMAX = -(-NFULL // NUM_WORKERS)
    UMAX = -(-TMAX // 2)
    mesh = plsc.VectorSubcoreMesh(
        core_axis_name="c", subcore_axis_name="s", num_cores=2, num_subcores=16
    )
    cp = pltpu.CompilerParams()
    if "needs_layout_passes" in pltpu.CompilerParams.__dataclass_fields__:
        cp = dataclasses.replace(cp, needs_layout_passes=False)

    @functools.partial(
        pl.kernel,
        out_type=(
            jax.ShapeDtypeStruct((K, M), jnp.float32),
            jax.ShapeDtypeStruct((K, 128), jnp.float32),
        ),
        mesh=mesh,
        compiler_params=cp,
        scratch_types=[
            pltpu.VMEM((3 * N,), jnp.float32),
            pltpu.VMEM((N,), jnp.float32),
            pltpu.VMEM((K, GRP), jnp.int32),
            pltpu.VMEM((K, GRP), jnp.int32),
            pltpu.VMEM((3, GRP), jnp.float32),
            pltpu.VMEM((3, GRP), jnp.float32),
            pltpu.VMEM((K, GRP), jnp.float32),
            pltpu.VMEM((K, GRP), jnp.float32),
            pltpu.SemaphoreType.DMA,
            pltpu.SemaphoreType.DMA,
            pltpu.SemaphoreType.DMA,
            pltpu.SemaphoreType.DMA,
            pltpu.SemaphoreType.DMA,
            pltpu.SemaphoreType.DMA,
        ],
    )
    def sc_kernel(q_hbm, i_hbm, qt_hbm, it_hbm, p_hbm, c_hbm, o_hbm, ot_hbm,
                  tpos, tcoef, ib0, ib1, qb0, qb1, ob0, ob1,
                  is0, is1, qs0, qs1, os0, os1):
        wid = lax.axis_index("s") * 2 + lax.axis_index("c")
        ibuf, qbuf, obuf = (ib0, ib1), (qb0, qb1), (ob0, ob1)
        isem, qsem, osem = (is0, is1), (qs0, qs1), (os0, os1)
        tp_desc = pltpu.async_copy(p_hbm, tpos, os0)
        tc_desc = pltpu.async_copy(c_hbm, tcoef, os1)

        def compute(ib, qb, ob, width):
            @plsc.parallel_loop(0, width // LANES, unroll=4)
            def _lane_group(l):
                lane = lax.iota(jnp.int32, LANES) + l * LANES

                def row(k):
                    return jnp.full((LANES,), k, jnp.int32)

                qx = plsc.load_gather(qb, [row(0), lane])
                qy = plsc.load_gather(qb, [row(1), lane])
                qz = plsc.load_gather(qb, [row(2), lane])
                acc = jnp.full((LANES,), 1e-8, jnp.float32)
                ws = []
                for k in range(K):
                    idxv = plsc.load_gather(ib, [row(k), lane])
                    i3 = idxv * 3
                    px = plsc.load_gather(tpos, [i3])
                    py = plsc.load_gather(tpos, [i3 + 1])
                    pz = plsc.load_gather(tpos, [i3 + 2])
                    cc = plsc.load_gather(tcoef, [idxv])
                    dx = px - qx
                    dy = py - qy
                    dz = pz - qz
                    d2 = dx * dx + dy * dy + dz * dz
                    w = jnp.exp(d2 * cc)
                    acc = acc + w
                    ws.append(w)
                r = jnp.full((LANES,), 1.0, jnp.float32) / acc
                for k in range(K):
                    plsc.store_scatter(ob, [row(k), lane], ws[k] * r)

        def start_in(g, b):
            m0 = g * GRP
            pltpu.async_copy(i_hbm.at[:, pl.ds(m0, GRP)], ibuf[b], isem[b])
            pltpu.async_copy(q_hbm.at[:, pl.ds(m0, GRP)], qbuf[b], qsem[b])

        def wait_in(b):
            pltpu.make_async_copy(i_hbm.at[:, pl.ds(0, GRP)], ibuf[b], isem[b]).wait()
            pltpu.make_async_copy(q_hbm.at[:, pl.ds(0, GRP)], qbuf[b], qsem[b]).wait()

        def wait_out(b):
            pltpu.make_async_copy(obuf[b], o_hbm.at[:, pl.ds(0, GRP)], osem[b]).wait()

        start_in(wid, 0)
        tp_desc.wait()
        tc_desc.wait()

        @pl.loop(0, UMAX)
        def _pair(u):
            t0 = u * 2
            for b in (0, 1):
                t = t0 + b
                g = wid + t * NUM_WORKERS

                @pl.when(g < NFULL)
                def _phase():
                    gn = g + NUM_WORKERS

                    @pl.when(gn < NFULL)
                    def _prefetch():
                        start_in(gn, 1 - b)

                    wait_in(b)

                    @pl.when(t >= 2)
                    def _drain_prev():
                        wait_out(b)

                    compute(ibuf[b], qbuf[b], obuf[b], GRP)
                    pltpu.async_copy(
                        obuf[b], o_hbm.at[:, pl.ds(g * GRP, GRP)], osem[b])

        nmine = (NFULL - wid + NUM_WORKERS - 1) // NUM_WORKERS
        for b in (0, 1):
            @pl.when(nmine >= b + 1)
            def _drain():
                wait_out(b)

        @pl.when(wid == 0)
        def _tail():
            pltpu.sync_copy(it_hbm, ib0.at[:, pl.ds(0, 128)])
            pltpu.sync_copy(qt_hbm, qb0.at[:, pl.ds(0, 128)])
            compute(ib0, qb0, ob0, 128)
            pltpu.sync_copy(ob0.at[:, pl.ds(0, 128)], ot_hbm)

    return sc_kernel


def kernel(query_points, k_nearest_indices, positions, log_radii):
    M, K = k_nearest_indices.shape
    N = positions.shape[0]
    TAIL = M % 128
    MAIN = M - TAIL

    coef = pl.pallas_call(
        _radii_coef_kernel,
        out_shape=jax.ShapeDtypeStruct(log_radii.shape, jnp.float32),
    )(log_radii.astype(jnp.float32))

    q_t = query_points.astype(jnp.float32).T
    i_t = k_nearest_indices.astype(jnp.int32).T
    q_tail = lax.dynamic_slice(q_t, (0, M - 128), (3, 128))
    i_tail = lax.dynamic_slice(i_t, (0, M - 128), (K, 128))

    out_t, out_tail = _make_sc_kernel(M, K, N, MAIN)(
        q_t,
        i_t,
        q_tail,
        i_tail,
        positions.astype(jnp.float32).reshape(-1),
        coef.reshape(-1),
    )
    if TAIL:
        out_t = lax.dynamic_update_slice(out_t, out_tail, (0, M - 128))
    return out_t.T
